# baseline (speedup 1.0000x reference)
import jax
import jax.numpy as jnp
from jax.experimental import pallas as pl
from jax.experimental.pallas import tpu as pltpu

_N = 64 * 64
_PAD = 128
_XC = _N + 2 * _PAD
_TILE = 2048
_NT = _N // _TILE
_RSQ = 0.9999950000374997


def _body(x_ref, w1_ref, gc_ref, bc_ref, gr_ref, br_ref,
          wc2_ref, bc2_ref, wr2_ref, br2_ref, cls_ref, reg_ref,
          xpad, xcol, svec, bvec):
    i = pl.program_id(0)

    @pl.when(i == 0)
    def _init():
        xpad[:, :_PAD] = jnp.zeros((256, _PAD), jnp.bfloat16)
        xpad[:, _N + _PAD:] = jnp.zeros((256, _PAD), jnp.bfloat16)
        xpad[:, _PAD:_N + _PAD] = x_ref[:, :].astype(jnp.bfloat16)
        xv = xpad[:, :]
        lane = jax.lax.broadcasted_iota(jnp.int32, (1, _N), 1)
        m0 = (lane % 64 != 0).astype(jnp.bfloat16)
        m2 = (lane % 64 != 63).astype(jnp.bfloat16)
        for k in range(9):
            dy, dx = divmod(k, 3)
            off = _PAD + (dy - 1) * 64 + (dx - 1)
            p = jax.lax.slice(xv, (0, off), (256, off + _N))
            if dx == 0:
                p = p * m0
            elif dx == 2:
                p = p * m2
            xcol[k * 256:(k + 1) * 256, :] = p
        svec[:256] = gc_ref[0].reshape(256, 1) * _RSQ
        svec[256:] = gr_ref[0].reshape(256, 1) * _RSQ
        bvec[:256] = bc_ref[0].reshape(256, 1)
        bvec[256:] = br_ref[0].reshape(256, 1)

    xs9 = xcol[:, pl.ds(i * _TILE, _TILE)]
    acc = jax.lax.dot_general(
        w1_ref[:, :], xs9, (((1,), (0,)), ((), ())),
        preferred_element_type=jnp.float32)
    acc = acc * svec[:, :1] + bvec[:, :1]
    h = (acc * jax.nn.sigmoid(acc)).astype(jnp.bfloat16)
    cls_ref[:, :] = jax.lax.dot_general(
        wc2_ref[:, :].astype(jnp.bfloat16), h[:256], (((1,), (0,)), ((), ())),
        preferred_element_type=jnp.float32) + bc2_ref[0].reshape(80, 1)
    reg_ref[:, :] = jax.lax.dot_general(
        wr2_ref[:, :].astype(jnp.bfloat16), h[256:], (((1,), (0,)), ((), ())),
        preferred_element_type=jnp.float32) + br2_ref[0].reshape(68, 1)


def kernel(feats, strides, training, cls_w1, cls_gamma, cls_beta, cls_w2,
           cls_b2, reg_w1, reg_gamma, reg_beta, reg_w2, reg_b2):
    w1 = jnp.concatenate([cls_w1, reg_w1], axis=0).astype(jnp.bfloat16)
    w1 = w1.reshape(512, 256, 9).transpose(0, 2, 1).reshape(512, 2304)
    full = lambda *dims: pl.BlockSpec(dims, lambda i: tuple(0 for _ in dims))
    cls_flat, reg_flat = pl.pallas_call(
        _body,
        grid=(_NT,),
        in_specs=[
            full(256, _N),
            full(512, 2304),
            full(1, 256), full(1, 256), full(1, 256), full(1, 256),
            full(80, 256), full(1, 80), full(68, 256), full(1, 68),
        ],
        out_specs=[
            pl.BlockSpec((80, _TILE), lambda i: (0, i)),
            pl.BlockSpec((68, _TILE), lambda i: (0, i)),
        ],
        out_shape=[
            jax.ShapeDtypeStruct((80, _N), jnp.float32),
            jax.ShapeDtypeStruct((68, _N), jnp.float32),
        ],
        scratch_shapes=[
            pltpu.VMEM((256, _XC), jnp.bfloat16),
            pltpu.VMEM((2304, _N), jnp.bfloat16),
            pltpu.VMEM((512, 1), jnp.float32),
            pltpu.VMEM((512, 1), jnp.float32),
        ],
        compiler_params=pltpu.CompilerParams(
            dimension_semantics=("arbitrary",)),
    )(feats.reshape(256, _N), w1, cls_gamma.reshape(1, 256),
      cls_beta.reshape(1, 256), reg_gamma.reshape(1, 256),
      reg_beta.reshape(1, 256), cls_w2.reshape(80, 256),
      cls_b2.reshape(1, 80), reg_w2.reshape(68, 256),
      reg_b2.reshape(1, 68))
    return (cls_flat.reshape(1, 80, 64, 64), reg_flat.reshape(1, 68, 64, 64))


# speedup vs baseline: 1.0989x; 1.0989x over previous
import jax
import jax.numpy as jnp
from jax.experimental import pallas as pl
from jax.experimental.pallas import tpu as pltpu

_N = 64 * 64
_PAD = 128
_XC = _N + 2 * _PAD
_TILE = 2048
_NT = _N // _TILE
_RSQ = 0.9999950000374997


def _body(x_ref, w1_ref, gc_ref, bc_ref, gr_ref, br_ref,
          wc2_ref, bc2_ref, wr2_ref, br2_ref, cls_ref, reg_ref,
          xq, svec, bvec):
    i = pl.program_id(0)

    @pl.when(i == 0)
    def _init():
        xv = x_ref[:, :]
        lane = jax.lax.broadcasted_iota(jnp.int32, (1, _N), 1)
        z1 = jnp.zeros((256, 1), jnp.float32)
        m0 = (lane % 64 != 0).astype(jnp.float32)
        m2 = (lane % 64 != 63).astype(jnp.float32)
        planes = (
            jnp.concatenate([z1, xv[:, :_N - 1]], axis=1) * m0,
            xv,
            jnp.concatenate([xv[:, 1:], z1], axis=1) * m2,
        )
        for d in range(3):
            xq[d, :, :_PAD] = jnp.zeros((256, _PAD), jnp.bfloat16)
            xq[d, :, _N + _PAD:] = jnp.zeros((256, _PAD), jnp.bfloat16)
            xq[d, :, _PAD:_N + _PAD] = planes[d].astype(jnp.bfloat16)
        svec[:256] = gc_ref[0].reshape(256, 1) * _RSQ
        svec[256:] = gr_ref[0].reshape(256, 1) * _RSQ
        bvec[:256] = bc_ref[0].reshape(256, 1)
        bvec[256:] = br_ref[0].reshape(256, 1)

    j0 = i * _TILE
    xw = (xq[0, :, pl.ds(j0, _TILE + 2 * _PAD)],
          xq[1, :, pl.ds(j0, _TILE + 2 * _PAD)],
          xq[2, :, pl.ds(j0, _TILE + 2 * _PAD)])
    acc = jnp.zeros((512, _TILE), jnp.float32)
    for k in range(9):
        dy, dx = divmod(k, 3)
        off = _PAD + (dy - 1) * 64
        xs = jax.lax.slice(xw[dx], (0, off), (256, off + _TILE))
        acc = acc + jax.lax.dot_general(
            w1_ref[k], xs, (((1,), (0,)), ((), ())),
            preferred_element_type=jnp.float32)
    acc = acc * svec[:, :1] + bvec[:, :1]
    h = (acc * jax.nn.sigmoid(acc)).astype(jnp.bfloat16)
    cls_ref[:, :] = jax.lax.dot_general(
        wc2_ref[:, :].astype(jnp.bfloat16), h[:256], (((1,), (0,)), ((), ())),
        preferred_element_type=jnp.float32) + bc2_ref[0].reshape(80, 1)
    reg_ref[:, :] = jax.lax.dot_general(
        wr2_ref[:, :].astype(jnp.bfloat16), h[256:], (((1,), (0,)), ((), ())),
        preferred_element_type=jnp.float32) + br2_ref[0].reshape(68, 1)


def kernel(feats, strides, training, cls_w1, cls_gamma, cls_beta, cls_w2,
           cls_b2, reg_w1, reg_gamma, reg_beta, reg_w2, reg_b2):
    w1 = jnp.concatenate([cls_w1, reg_w1], axis=0).astype(jnp.bfloat16)
    w1 = w1.reshape(512, 256, 9).transpose(2, 0, 1)
    full = lambda *dims: pl.BlockSpec(dims, lambda i: tuple(0 for _ in dims))
    cls_flat, reg_flat = pl.pallas_call(
        _body,
        grid=(_NT,),
        in_specs=[
            full(256, _N),
            full(9, 512, 256),
            full(1, 256), full(1, 256), full(1, 256), full(1, 256),
            full(80, 256), full(1, 80), full(68, 256), full(1, 68),
        ],
        out_specs=[
            pl.BlockSpec((80, _TILE), lambda i: (0, i)),
            pl.BlockSpec((68, _TILE), lambda i: (0, i)),
        ],
        out_shape=[
            jax.ShapeDtypeStruct((80, _N), jnp.float32),
            jax.ShapeDtypeStruct((68, _N), jnp.float32),
        ],
        scratch_shapes=[
            pltpu.VMEM((3, 256, _XC), jnp.bfloat16),
            pltpu.VMEM((512, 1), jnp.float32),
            pltpu.VMEM((512, 1), jnp.float32),
        ],
        compiler_params=pltpu.CompilerParams(
            dimension_semantics=("arbitrary",)),
    )(feats.reshape(256, _N), w1, cls_gamma.reshape(1, 256),
      cls_beta.reshape(1, 256), reg_gamma.reshape(1, 256),
      reg_beta.reshape(1, 256), cls_w2.reshape(80, 256),
      cls_b2.reshape(1, 80), reg_w2.reshape(68, 256),
      reg_b2.reshape(1, 68))
    return (cls_flat.reshape(1, 80, 64, 64), reg_flat.reshape(1, 68, 64, 64))
